# baseline (speedup 1.0000x reference)
import jax
import jax.numpy as jnp
from jax import lax
from jax.experimental import pallas as pl
from jax.experimental.pallas import tpu as pltpu
from jax.experimental.pallas import tpu_sc as plsc

TGB = 128
B, H, DENSE, NS, G, CAP, V = 16, 256, 512, 8, 3, 192, 100000
T = CAP * 2 - 1
TP = 384
TOUT = NS + T
RB = 4
SB = B // RB
TGP = 256
GP = 8

_SC_NC, _SC_NS = 2, 16
_NW = _SC_NC * _SC_NS
_TOK_ROWS = 4 * B * CAP
_ROWS_PER_W = _TOK_ROWS // _NW
_CHUNK = 128
_NCH = _ROWS_PER_W // _CHUNK


def _sc_gather_body(tok_idx, tok_emb, out, i0, i1, i2, b0, b1, b2,
                    isem, g0, g1, g2, w0, w1, w2):
    wid = lax.axis_index("s") * _SC_NC + lax.axis_index("c")
    base = wid * _ROWS_PER_W
    idxs, bufs, gsem, wsem = [i0, i1, i2], [b0, b1, b2], [g0, g1, g2], [w0, w1, w2]
    ih = [pltpu.async_copy(tok_idx.at[pl.ds(base + k * _CHUNK, _CHUNK)],
                           idxs[k], isem) for k in range(_NCH)]
    hg = [None] * _NCH
    for k in range(_NCH):
        ih[k].wait()
        hg[k] = pltpu.async_copy(tok_emb.at[idxs[k]], bufs[k], gsem[k])
    hw = [None] * _NCH
    for k in range(_NCH):
        hg[k].wait()
        hw[k] = pltpu.async_copy(bufs[k], out.at[pl.ds(base + k * _CHUNK,
                                                       _CHUNK)], wsem[k])
    for k in range(_NCH):
        hw[k].wait()


def _sc_gather(*args):
    fn = pl.kernel(
        _sc_gather_body,
        mesh=plsc.VectorSubcoreMesh(core_axis_name="c", subcore_axis_name="s"),
        out_type=jax.ShapeDtypeStruct((_TOK_ROWS, H), jnp.float32),
        scratch_types=(
            [pltpu.VMEM((_CHUNK,), jnp.int32) for _ in range(_NCH)]
            + [pltpu.VMEM((_CHUNK, H), jnp.float32) for _ in range(_NCH)]
            + [pltpu.SemaphoreType.DMA for _ in range(1 + 2 * _NCH)]
        ),
    )
    return fn(*args)


def _ln(x, g, b, eps=1e-5):
    m = jnp.mean(x, axis=-1, keepdims=True)
    v = jnp.mean((x - m) * (x - m), axis=-1, keepdims=True)
    return (x - m) * lax.rsqrt(v + eps) * g + b


def _silu(x):
    return x * jax.nn.sigmoid(x)


def _tc_body(len_ref, ev4_ref, g_ref, th_ref,
             dense_ref, user_ref, ctx_ref, cpost_ref, cauth_ref, cand_ref,
             Wd_ref, bd_ref, lndg_ref, lndb_ref, ln6g_ref, ln6b_ref,
             Wa1_ref, ba1_ref, Wa2_ref, ba2_ref, gpe_ref,
             lneg_ref, lneb_ref, We1_ref, be1_ref, We2_ref, be2_ref,
             tg_ref, sg_ref, spe_ref, sep_ref, mlt_ref,
             out_ns_ref, out_tok_ref, out_mask_ref):
    b = pl.program_id(0)

    @pl.when(b == 0)
    def _ns_path():
        x = jnp.dot(dense_ref[...], Wd_ref[...],
                    preferred_element_type=jnp.float32) + bd_ref[...]
        ds = _silu(_ln(x, lndg_ref[...], lndb_ref[...]))
        fused = jnp.concatenate(
            [ds, user_ref[...], ctx_ref[...], cpost_ref[...],
             cauth_ref[...], cand_ref[...]], axis=1)
        h = _ln(fused, ln6g_ref[...], ln6b_ref[...])
        h = _silu(jnp.dot(h.astype(jnp.bfloat16), Wa1_ref[...],
                          preferred_element_type=jnp.float32) + ba1_ref[...])
        out_ns_ref[...] = (jnp.dot(h.astype(jnp.bfloat16), Wa2_ref[...],
                                   preferred_element_type=jnp.float32)
                           + ba2_ref[...] + gpe_ref[...])

    th_chunks, gh_chunks = [], []
    for rr_ in range(RB):
        g_r = g_ref[rr_]
        th_r = th_ref[rr_]
        th_r = jnp.minimum(jnp.maximum(th_r, 0), TGB)
        vg = lax.broadcasted_iota(jnp.int32, (TGP, CAP), 0)
        Nth = (vg == jnp.broadcast_to(th_r, (TGP, CAP))).astype(jnp.float32)
        th_chunks.append(lax.dot_general(
            Nth, tg_ref[...], (((0,), (0,)), ((), ())),
            preferred_element_type=jnp.float32))
        vg2 = lax.broadcasted_iota(jnp.int32, (GP, CAP), 0)
        Ng = (vg2 == jnp.broadcast_to(g_r, (GP, CAP))).astype(jnp.float32)
        gh_chunks.append(lax.dot_general(
            Ng, sg_ref[...], (((0,), (0,)), ((), ())),
            preferred_element_type=jnp.float32))
    th_chunk = jnp.concatenate(th_chunks, axis=0)
    gh_chunk = jnp.concatenate(gh_chunks, axis=0)

    M = RB * CAP
    xs = [ev4_ref[c].reshape(M, H) for c in range(4)] + [th_chunk, gh_chunk]
    s = xs[0].sum(axis=1, keepdims=True)
    ss = (xs[0] * xs[0]).sum(axis=1, keepdims=True)
    for c in range(1, 6):
        s = s + xs[c].sum(axis=1, keepdims=True)
        ss = ss + (xs[c] * xs[c]).sum(axis=1, keepdims=True)
    inv = 1.0 / (6 * H)
    m = s * inv
    var = ss * inv - m * m
    r = lax.rsqrt(var + 1e-5)
    bf16 = jnp.bfloat16
    acc = None
    for c in range(6):
        y = ((xs[c] - m) * r * lneg_ref[c] + lneb_ref[c]).astype(bf16)
        p = jnp.dot(y, We1_ref[c],
                    preferred_element_type=jnp.float32)
        acc = p if acc is None else acc + p
    h = _silu(acc + be1_ref[...])
    evs = jnp.dot(h.astype(bf16), We2_ref[...],
                  preferred_element_type=jnp.float32) + be2_ref[...]

    for rr_ in range(RB):
        g_r = g_ref[rr_]
        ev_row = evs[rr_ * CAP:(rr_ + 1) * CAP]
        L = len_ref[b * RB + rr_]
        ii = lax.broadcasted_iota(jnp.int32, (1, CAP), 1)
        g_next = pltpu.roll(g_r, CAP - 1, 1)
        sep = ((ii + 1) < L) & (g_r != g_next)
        sep_f = sep.astype(jnp.float32)
        S = jnp.sum(sep.astype(jnp.int32))
        total = L + S
        base = T - total
        nsepb = jnp.dot(sep_f, mlt_ref[...],
                        preferred_element_type=jnp.float32)
        pos_tok = base + ii + nsepb.astype(jnp.int32)
        valid = ii < L
        tt = lax.broadcasted_iota(jnp.int32, (TP, CAP), 0)
        pt = jnp.broadcast_to(pos_tok, (TP, CAP))
        P = ((tt == pt) & jnp.broadcast_to(valid, (TP, CAP))).astype(bf16)
        merged = jnp.dot(P, ev_row.astype(bf16),
                         preferred_element_type=jnp.float32)
        sepflag = jnp.sum(
            ((tt == pt + 1) & jnp.broadcast_to(sep, (TP, CAP))).astype(
                jnp.float32), axis=1, keepdims=True)
        merged = merged + sepflag * sep_ref[...]
        merged = merged + spe_ref[...]
        tcol = lax.broadcasted_iota(jnp.int32, (TP, 1), 0)
        merged = merged * (tcol >= base).astype(jnp.float32)
        out_tok_ref[rr_] = merged[:T]
        to = lax.broadcasted_iota(jnp.int32, (1, TOUT), 1)
        out_mask_ref[rr_] = ((to < NS) | (to >= (base + NS))).astype(jnp.int32)


def _tc_specs():
    full = lambda shape: pl.BlockSpec(shape, lambda b, L: (0,) * len(shape))
    in_specs = [
        pl.BlockSpec((4, RB, CAP, H), lambda b, L: (0, b, 0, 0)),
        pl.BlockSpec((RB, 1, CAP), lambda b, L: (b, 0, 0)),
        pl.BlockSpec((RB, 1, CAP), lambda b, L: (b, 0, 0)),
        full((B, DENSE)), full((B, H)), full((B, H)), full((B, H)),
        full((B, H)), full((B, H)),
        full((DENSE, H)), full((1, H)), full((1, H)), full((1, H)),
        full((1, 6 * H)), full((1, 6 * H)),
        full((6 * H, 4 * H)), full((1, 4 * H)),
        full((4 * H, NS * H)), full((1, NS * H)), full((1, NS * H)),
        full((6, 1, H)), full((6, 1, H)),
        full((6, H, 4 * H)), full((1, 4 * H)),
        full((4 * H, H)), full((1, H)),
        full((TGP, H)), full((GP, H)),
        full((TP, H)), full((1, H)), full((CAP, CAP)),
    ]
    out_specs = [
        pl.BlockSpec((B, NS * H), lambda b, L: (0, 0)),
        pl.BlockSpec((RB, T, H), lambda b, L: (b, 0, 0)),
        pl.BlockSpec((RB, 1, TOUT), lambda b, L: (b, 0, 0)),
    ]
    out_shape = [
        jax.ShapeDtypeStruct((B, NS * H), jnp.float32),
        jax.ShapeDtypeStruct((B, T, H), jnp.float32),
        jax.ShapeDtypeStruct((B, 1, TOUT), jnp.int32),
    ]
    scratch = []
    return in_specs, out_specs, out_shape, scratch


def _tc_forward(lengths, ev4, g3, th3, dense_features, user_tokens,
                context_tokens, candidate_post_tokens,
                candidate_author_tokens, candidate_tokens, Wd, bd, lndg, lndb,
                ln6g, ln6b, Wa1, ba1, Wa2r, ba2r, gpe, lneg, lneb, We1r, be1,
                We2, be2, tgp, sgp, spe, sep, mlt, interpret=False):
    in_specs, out_specs, out_shape, scratch = _tc_specs()
    grid_spec = pltpu.PrefetchScalarGridSpec(
        num_scalar_prefetch=1,
        grid=(SB,),
        in_specs=in_specs,
        out_specs=out_specs,
        scratch_shapes=scratch,
    )
    return pl.pallas_call(
        _tc_body,
        grid_spec=grid_spec,
        out_shape=out_shape,
        interpret=interpret,
    )(lengths, ev4, g3, th3, dense_features, user_tokens, context_tokens,
      candidate_post_tokens, candidate_author_tokens, candidate_tokens,
      Wd, bd, lndg, lndb, ln6g, ln6b, Wa1, ba1, Wa2r, ba2r, gpe,
      lneg, lneb, We1r, be1, We2, be2, tgp, sgp, spe, sep, mlt)


def kernel(dense_features, user_tokens, context_tokens, candidate_tokens,
           candidate_post_tokens, candidate_author_tokens, history_tokens,
           history_post_tokens, history_author_tokens, history_action_tokens,
           history_time_gap, history_group_ids, history_lengths,
           W_dense, b_dense, ln_d_g, ln_d_b, ln6_g, ln6_b, W_a1, b_a1,
           W_a2, b_a2, group_pos_emb, token_emb, ln_e_g, ln_e_b, W_e1, b_e1,
           W_e2, b_e2, time_gap_emb, seq_group_emb, seq_pos_emb, sep_token):
    i32 = jnp.int32
    tok_idx = jnp.concatenate(
        [history_tokens.reshape(-1), history_post_tokens.reshape(-1),
         history_author_tokens.reshape(-1),
         history_action_tokens.reshape(-1)]).astype(i32)
    ev_rows = _sc_gather(tok_idx, token_emb)
    ev4 = ev_rows.reshape(4, B, CAP, H)

    g3 = history_group_ids.astype(i32).reshape(B, 1, CAP)
    th3 = history_time_gap.astype(i32).reshape(B, 1, CAP)
    spe = jnp.pad(seq_pos_emb, ((0, TP - T), (0, 0)))
    tgp = jnp.pad(time_gap_emb, ((0, TGP - (TGB + 1)), (0, 0)))
    sgp = jnp.pad(seq_group_emb, ((0, GP - (G + 1)), (0, 0)))
    mlt = jnp.triu(jnp.ones((CAP, CAP), jnp.float32), 1)
    ns_flat, merged, mask3 = _tc_forward(
        history_lengths.astype(i32), ev4, g3, th3,
        dense_features, user_tokens, context_tokens, candidate_post_tokens,
        candidate_author_tokens, candidate_tokens,
        W_dense, b_dense.reshape(1, H), ln_d_g.reshape(1, H),
        ln_d_b.reshape(1, H), ln6_g.reshape(1, 6 * H), ln6_b.reshape(1, 6 * H),
        W_a1.astype(jnp.bfloat16), b_a1.reshape(1, 4 * H),
        W_a2.astype(jnp.bfloat16),
        b_a2.reshape(1, NS * H), group_pos_emb.reshape(1, NS * H),
        ln_e_g.reshape(6, 1, H), ln_e_b.reshape(6, 1, H),
        W_e1.reshape(6, H, 4 * H).astype(jnp.bfloat16),
        b_e1.reshape(1, 4 * H), W_e2.astype(jnp.bfloat16),
        b_e2.reshape(1, H), tgp, sgp, spe, sep_token.reshape(1, H), mlt)
    tokens = jnp.concatenate([ns_flat.reshape(B, NS, H), merged], axis=1)
    return tokens, mask3.reshape(B, TOUT) > 0


# speedup vs baseline: 1.2305x; 1.2305x over previous
import jax
import jax.numpy as jnp
from jax import lax
from jax.experimental import pallas as pl
from jax.experimental.pallas import tpu as pltpu
from jax.experimental.pallas import tpu_sc as plsc

TGB = 128
B, H, DENSE, NS, G, CAP, V = 16, 256, 512, 8, 3, 192, 100000
T = CAP * 2 - 1
TP = 384
TOUT = NS + T
RB = 4
SB = B // RB

_SC_NC, _SC_NS = 2, 16
_NW = _SC_NC * _SC_NS
_TOK_ROWS = 4 * B * CAP
_ROWS_PER_W = _TOK_ROWS // _NW
_CHUNK = 128
_NCH = _ROWS_PER_W // _CHUNK


def _sc_gather_body(tok_idx, tok_emb, out, i0, i1, i2, b0, b1, b2,
                    isem, g0, g1, g2, w0, w1, w2):
    wid = lax.axis_index("s") * _SC_NC + lax.axis_index("c")
    base = wid * _ROWS_PER_W
    idxs, bufs = [i0, i1, i2], [b0, b1, b2]
    gsem, wsem = [g0, g1, g2], [w0, w1, w2]
    ih = [pltpu.async_copy(tok_idx.at[pl.ds(base + k * _CHUNK, _CHUNK)],
                           idxs[k], isem) for k in range(_NCH)]
    hg = [None] * _NCH
    for k in range(_NCH):
        ih[k].wait()
        hg[k] = pltpu.async_copy(tok_emb.at[idxs[k]], bufs[k], gsem[k])
    hw = [None] * _NCH
    for k in range(_NCH):
        hg[k].wait()
        hw[k] = pltpu.async_copy(bufs[k], out.at[pl.ds(base + k * _CHUNK,
                                                       _CHUNK)], wsem[k])
    for k in range(_NCH):
        hw[k].wait()


def _sc_gather(*args):
    fn = pl.kernel(
        _sc_gather_body,
        mesh=plsc.VectorSubcoreMesh(core_axis_name="c", subcore_axis_name="s"),
        out_type=jax.ShapeDtypeStruct((_TOK_ROWS, H), jnp.float32),
        scratch_types=(
            [pltpu.VMEM((_CHUNK,), jnp.int32) for _ in range(_NCH)]
            + [pltpu.VMEM((_CHUNK, H), jnp.float32) for _ in range(_NCH)]
            + [pltpu.SemaphoreType.DMA for _ in range(1 + 2 * _NCH)]
        ),
    )
    return fn(*args)


def _ln(x, g, b, eps=1e-5):
    m = jnp.mean(x, axis=-1, keepdims=True)
    v = jnp.mean((x - m) * (x - m), axis=-1, keepdims=True)
    return (x - m) * lax.rsqrt(v + eps) * g + b


def _silu(x):
    return x * jax.nn.sigmoid(x)


def _tc_body(len_ref, ev4_ref, g_ref, th_ref,
             dense_ref, user_ref, ctx_ref, cpost_ref, cauth_ref, cand_ref,
             Wd_ref, bd_ref, lndg_ref, lndb_ref, ln6g_ref, ln6b_ref,
             Wa1_ref, ba1_ref, Wa2_ref, ba2_ref, gpe_ref,
             lneg_ref, lneb_ref, We1_ref, be1_ref, We2_ref, be2_ref,
             tg_ref, sg_ref, spe_ref, sep_ref,
             out_tok_ref, out_mask_ref, ns_scr):
    b = pl.program_id(0)
    bf16 = jnp.bfloat16

    @pl.when(b == 0)
    def _ns_path():
        x = jnp.dot(dense_ref[...], Wd_ref[...],
                    preferred_element_type=jnp.float32) + bd_ref[...]
        ds = _silu(_ln(x, lndg_ref[...], lndb_ref[...]))
        fused = jnp.concatenate(
            [ds, user_ref[...], ctx_ref[...], cpost_ref[...],
             cauth_ref[...], cand_ref[...]], axis=1)
        hq = _ln(fused, ln6g_ref[...], ln6b_ref[...])
        hq = _silu(jnp.dot(hq, Wa1_ref[...],
                           preferred_element_type=jnp.float32) + ba1_ref[...])
        for k in range(NS):
            ns_scr[k] = (jnp.dot(hq, Wa2_ref[:, k * H:(k + 1) * H],
                                 preferred_element_type=jnp.float32)
                         + ba2_ref[k:k + 1, :] + gpe_ref[k:k + 1, :])

    th_chunks, gh_chunks = [], []
    for rr_ in range(RB):
        g_r = g_ref[rr_]
        th_r = th_ref[rr_]
        th_r = jnp.minimum(jnp.maximum(th_r, 0), TGB)
        vg = lax.broadcasted_iota(jnp.int32, (TGB + 1, CAP), 0)
        Nth = (vg == jnp.broadcast_to(th_r, (TGB + 1, CAP))).astype(
            jnp.float32)
        th_chunks.append(lax.dot_general(
            Nth, tg_ref[...], (((0,), (0,)), ((), ())),
            preferred_element_type=jnp.float32))
        vg2 = lax.broadcasted_iota(jnp.int32, (G + 1, CAP), 0)
        Ng = (vg2 == jnp.broadcast_to(g_r, (G + 1, CAP))).astype(jnp.float32)
        gh_chunks.append(lax.dot_general(
            Ng, sg_ref[...], (((0,), (0,)), ((), ())),
            preferred_element_type=jnp.float32))
    th_chunk = jnp.concatenate(th_chunks, axis=0)
    gh_chunk = jnp.concatenate(gh_chunks, axis=0)

    M = RB * CAP
    xs = [ev4_ref[c].reshape(M, H) for c in range(4)] + [th_chunk, gh_chunk]
    s = xs[0].sum(axis=1, keepdims=True)
    ss = (xs[0] * xs[0]).sum(axis=1, keepdims=True)
    for c in range(1, 6):
        s = s + xs[c].sum(axis=1, keepdims=True)
        ss = ss + (xs[c] * xs[c]).sum(axis=1, keepdims=True)
    inv = 1.0 / (6 * H)
    m = s * inv
    var = ss * inv - m * m
    r = lax.rsqrt(var + 1e-5)
    acc = None
    for c in range(6):
        y = ((xs[c] - m) * r * lneg_ref[c] + lneb_ref[c]).astype(bf16)
        p = jnp.dot(y, We1_ref[c].astype(bf16),
                    preferred_element_type=jnp.float32)
        acc = p if acc is None else acc + p
    h = _silu(acc + be1_ref[...])
    evs = jnp.dot(h.astype(bf16), We2_ref[...].astype(bf16),
                  preferred_element_type=jnp.float32) + be2_ref[...]

    rrm = lax.broadcasted_iota(jnp.int32, (CAP, CAP), 0)
    ccm = lax.broadcasted_iota(jnp.int32, (CAP, CAP), 1)
    mlt = (rrm < ccm).astype(jnp.float32)
    for rr_ in range(RB):
        g_r = g_ref[rr_]
        ev_row = evs[rr_ * CAP:(rr_ + 1) * CAP]
        L = len_ref[b * RB + rr_]
        ii = lax.broadcasted_iota(jnp.int32, (1, CAP), 1)
        g_next = pltpu.roll(g_r, CAP - 1, 1)
        sep = ((ii + 1) < L) & (g_r != g_next)
        sep_f = sep.astype(jnp.float32)
        S = jnp.sum(sep.astype(jnp.int32))
        total = L + S
        base = T - total
        nsepb = jnp.dot(sep_f, mlt, preferred_element_type=jnp.float32)
        pos_tok = base + ii + nsepb.astype(jnp.int32)
        valid = ii < L
        tt = lax.broadcasted_iota(jnp.int32, (TP, CAP), 0)
        pt = jnp.broadcast_to(pos_tok, (TP, CAP))
        P = ((tt == pt) & jnp.broadcast_to(valid, (TP, CAP))).astype(bf16)
        merged = jnp.dot(P, ev_row.astype(bf16),
                         preferred_element_type=jnp.float32)
        sepflag = jnp.sum(
            ((tt == pt + 1) & jnp.broadcast_to(sep, (TP, CAP))).astype(
                jnp.float32), axis=1, keepdims=True)
        merged = merged + sepflag * sep_ref[...]
        tcol = lax.broadcasted_iota(jnp.int32, (TP, 1), 0)
        live = (tcol >= base).astype(jnp.float32)
        out_row = (merged[:T] + spe_ref[...]) * live[:T]
        ns_row = ns_scr[:, pl.ds(b * RB + rr_, 1), :].reshape(NS, H)
        out_tok_ref[rr_] = jnp.concatenate([ns_row, out_row], axis=0)
        to = lax.broadcasted_iota(jnp.int32, (1, TOUT), 1)
        out_mask_ref[rr_] = ((to < NS) | (to >= (base + NS))).astype(jnp.int32)


def _tc_specs():
    full = lambda shape: pl.BlockSpec(shape, lambda b, L: (0,) * len(shape))
    in_specs = [
        pl.BlockSpec((4, RB, CAP, H), lambda b, L: (0, b, 0, 0)),
        pl.BlockSpec((RB, 1, CAP), lambda b, L: (b, 0, 0)),
        pl.BlockSpec((RB, 1, CAP), lambda b, L: (b, 0, 0)),
        full((B, DENSE)), full((B, H)), full((B, H)), full((B, H)),
        full((B, H)), full((B, H)),
        full((DENSE, H)), full((1, H)), full((1, H)), full((1, H)),
        full((1, 6 * H)), full((1, 6 * H)),
        full((6 * H, 4 * H)), full((1, 4 * H)),
        full((4 * H, NS * H)), full((NS, H)), full((NS, H)),
        full((6, 1, H)), full((6, 1, H)),
        full((6, H, 4 * H)), full((1, 4 * H)),
        full((4 * H, H)), full((1, H)),
        full((TGB + 1, H)), full((G + 1, H)),
        full((T, H)), full((1, H)),
    ]
    out_specs = [
        pl.BlockSpec((RB, TOUT, H), lambda b, L: (b, 0, 0)),
        pl.BlockSpec((RB, 1, TOUT), lambda b, L: (b, 0, 0)),
    ]
    out_shape = [
        jax.ShapeDtypeStruct((B, TOUT, H), jnp.float32),
        jax.ShapeDtypeStruct((B, 1, TOUT), jnp.int32),
    ]
    scratch = [pltpu.VMEM((NS, B, H), jnp.float32)]
    return in_specs, out_specs, out_shape, scratch


def _tc_forward(lengths, ev4, g3, th3, dense_features, user_tokens,
                context_tokens, candidate_post_tokens,
                candidate_author_tokens, candidate_tokens, Wd, bd, lndg, lndb,
                ln6g, ln6b, Wa1, ba1, Wa2, ba2, gpe, lneg, lneb, We1r, be1,
                We2, be2, tg, sg, spe, sep, interpret=False):
    in_specs, out_specs, out_shape, scratch = _tc_specs()
    grid_spec = pltpu.PrefetchScalarGridSpec(
        num_scalar_prefetch=1,
        grid=(SB,),
        in_specs=in_specs,
        out_specs=out_specs,
        scratch_shapes=scratch,
    )
    return pl.pallas_call(
        _tc_body,
        grid_spec=grid_spec,
        out_shape=out_shape,
        interpret=interpret,
    )(lengths, ev4, g3, th3, dense_features, user_tokens, context_tokens,
      candidate_post_tokens, candidate_author_tokens, candidate_tokens,
      Wd, bd, lndg, lndb, ln6g, ln6b, Wa1, ba1, Wa2, ba2, gpe,
      lneg, lneb, We1r, be1, We2, be2, tg, sg, spe, sep)


def kernel(dense_features, user_tokens, context_tokens, candidate_tokens,
           candidate_post_tokens, candidate_author_tokens, history_tokens,
           history_post_tokens, history_author_tokens, history_action_tokens,
           history_time_gap, history_group_ids, history_lengths,
           W_dense, b_dense, ln_d_g, ln_d_b, ln6_g, ln6_b, W_a1, b_a1,
           W_a2, b_a2, group_pos_emb, token_emb, ln_e_g, ln_e_b, W_e1, b_e1,
           W_e2, b_e2, time_gap_emb, seq_group_emb, seq_pos_emb, sep_token):
    i32 = jnp.int32
    tok_idx = jnp.concatenate(
        [history_tokens.reshape(-1), history_post_tokens.reshape(-1),
         history_author_tokens.reshape(-1),
         history_action_tokens.reshape(-1)]).astype(i32)
    ev_rows = _sc_gather(tok_idx, token_emb)
    ev4 = ev_rows.reshape(4, B, CAP, H)

    g3 = history_group_ids.astype(i32).reshape(B, 1, CAP)
    th3 = history_time_gap.astype(i32).reshape(B, 1, CAP)
    tokens, mask3 = _tc_forward(
        history_lengths.astype(i32), ev4, g3, th3,
        dense_features, user_tokens, context_tokens, candidate_post_tokens,
        candidate_author_tokens, candidate_tokens,
        W_dense, b_dense.reshape(1, H), ln_d_g.reshape(1, H),
        ln_d_b.reshape(1, H), ln6_g.reshape(1, 6 * H), ln6_b.reshape(1, 6 * H),
        W_a1, b_a1.reshape(1, 4 * H), W_a2,
        b_a2.reshape(NS, H), group_pos_emb,
        ln_e_g.reshape(6, 1, H), ln_e_b.reshape(6, 1, H),
        W_e1.reshape(6, H, 4 * H), b_e1.reshape(1, 4 * H), W_e2,
        b_e2.reshape(1, H), time_gap_emb, seq_group_emb, seq_pos_emb,
        sep_token.reshape(1, H))
    return tokens, mask3.reshape(B, TOUT) > 0
